# Initial kernel scaffold; baseline (speedup 1.0000x reference)
import jax
import jax.numpy as jnp
from jax.experimental import pallas as pl


def kernel(mem, strengths, idx, val, query):
    raise NotImplementedError("write your pallas kernel here")


# speedup vs baseline: 2.4862x; 2.4862x over previous
import functools

import jax
import jax.numpy as jnp
from jax import lax
from jax.experimental import pallas as pl
from jax.experimental.pallas import tpu as pltpu
from jax.experimental.pallas import tpu_sc as plsc

CAP = 1_000_000
D = 64
K = 8
NWR = 16384
NQ = 16
DECAY = 0.02

SP = 1_000_448
CH = SP // 16
QCH = CH // 4
WCH = NWR // 16

BLK = 8000
NB = CAP // BLK
NEG = -1e30


def _sc_mark(idx, expect):
    mesh = plsc.VectorSubcoreMesh(core_axis_name="c", subcore_axis_name="s")

    @functools.partial(
        pl.kernel,
        mesh=mesh,
        out_type=[
            jax.ShapeDtypeStruct((SP,), jnp.float32),
            jax.ShapeDtypeStruct((NWR,), jnp.float32),
        ],
        scratch_types=[
            pltpu.VMEM((QCH,), jnp.float32),
            pltpu.VMEM((NWR,), jnp.int32),
            pltpu.VMEM((NWR,), jnp.float32),
            pltpu.VMEM((WCH,), jnp.int32),
            pltpu.VMEM((WCH,), jnp.float32),
            pltpu.VMEM((WCH,), jnp.float32),
            pltpu.SemaphoreType.DMA,
        ],
    )
    def k(idx_hbm, exp_hbm, ord_hbm, valid_hbm,
          zbuf, idx_all, exp_all, idxc, gbuf, ebuf, sem):
        cid = lax.axis_index("c")
        sid = lax.axis_index("s")

        @pl.when(cid == 0)
        def _work():
            def zb(t, c):
                zbuf[pl.ds(t * 16, 16)] = jnp.zeros((16,), jnp.float32)
                return c
            lax.fori_loop(0, QCH // 16, zb, 0)
            base = sid * CH

            def cq(q, c):
                pltpu.sync_copy(zbuf, ord_hbm.at[pl.ds(base + q * QCH, QCH)])
                return c
            lax.fori_loop(0, 4, cq, 0)
            plsc.subcore_barrier()

            @pl.when(sid == 0)
            def _scatter():
                pltpu.sync_copy(idx_hbm, idx_all)
                pltpu.sync_copy(exp_hbm, exp_all)
                pltpu.async_copy(exp_all, ord_hbm.at[idx_all], sem).wait()
            plsc.subcore_barrier()

            wbase = sid * WCH
            pltpu.sync_copy(idx_hbm.at[pl.ds(wbase, WCH)], idxc)
            pltpu.async_copy(ord_hbm.at[idxc], gbuf, sem).wait()
            pltpu.sync_copy(exp_hbm.at[pl.ds(wbase, WCH)], ebuf)

            def vv(t, c):
                g = gbuf[pl.ds(t * 16, 16)]
                e = ebuf[pl.ds(t * 16, 16)]
                ebuf[pl.ds(t * 16, 16)] = jnp.where(
                    g == e, jnp.float32(1.0), jnp.float32(0.0))
                return c
            lax.fori_loop(0, WCH // 16, vv, 0)
            pltpu.sync_copy(ebuf, valid_hbm.at[pl.ds(wbase, WCH)])

    return k(idx, expect)


def _tc_body(query_ref, val_ref, valid_ref, mem_ref, wm_ref, st_ref,
             retr_ref, tops_ref, str3_ref, qn_ref, rv_ref, rr_ref):
    i = pl.program_id(0)
    wm = wm_ref[0, 0, :]
    str3_ref[0, 0, :] = jnp.where(
        wm > 0.5, jnp.float32(1.0 - DECAY), st_ref[0, 0, :] * (1.0 - DECAY))

    def extract8(ext, row_of):
        iota = lax.broadcasted_iota(jnp.int32, ext.shape, 1)
        vals, rows = [], []
        for _ in range(K):
            m = jnp.max(ext, axis=1)
            am = jnp.min(jnp.where(ext == m[:, None], iota, jnp.int32(1 << 30)),
                         axis=1)
            oh = (iota == am[:, None]).astype(jnp.float32)
            vals.append(m)
            rows.append(row_of(oh))
            ext = jnp.where(iota == am[:, None], NEG, ext)
        rv_ref[...] = jnp.stack(vals, axis=1)
        rr_ref[...] = jnp.stack(rows, axis=1)

    @pl.when(i == 0)
    def _init():
        q = query_ref[...]
        qn = q / (jnp.sqrt(jnp.sum(q * q, axis=1, keepdims=True)) + 1e-12)
        qn_ref[...] = qn
        val = val_ref[...]
        scale = 1.0 / (jnp.sqrt(jnp.sum(val * val, axis=1)) + 1e-12)
        ws = jnp.dot(qn, val.T, preferred_element_type=jnp.float32)
        ws = ws * scale[None, :]
        ws = jnp.where(valid_ref[0, :][None, :] > 0.5, ws, NEG)
        mem0 = mem_ref[...]
        sims0 = jnp.dot(qn, mem0.T, preferred_element_type=jnp.float32)
        sims0 = jnp.where(wm[None, :] > 0.5, NEG, sims0)
        ext = jnp.concatenate([sims0, ws], axis=1)

        def row_of(oh):
            return (jnp.dot(oh[:, :BLK], mem0,
                            preferred_element_type=jnp.float32)
                    + jnp.dot(oh[:, BLK:] * scale[None, :], val,
                              preferred_element_type=jnp.float32))
        extract8(ext, row_of)

    @pl.when(i > 0)
    def _scan():
        qn = qn_ref[...]
        memb = mem_ref[...]
        sims = jnp.dot(qn, memb.T, preferred_element_type=jnp.float32)
        sims = jnp.where(wm[None, :] > 0.5, NEG, sims)
        prev_v = rv_ref[...]
        prev_r = rr_ref[...]
        ext = jnp.concatenate([prev_v, sims], axis=1)

        def row_of(oh):
            row = jnp.dot(oh[:, K:], memb, preferred_element_type=jnp.float32)
            for t in range(K):
                row = row + oh[:, t][:, None] * prev_r[:, t, :]
            return row
        extract8(ext, row_of)

    @pl.when(i == NB - 1)
    def _fin():
        tv = rv_ref[...]
        e = jnp.exp(tv - jnp.max(tv, axis=1, keepdims=True))
        w = e / jnp.sum(e, axis=1, keepdims=True)
        tops_ref[...] = tv
        retr_ref[...] = jnp.sum(w[:, :, None] * rr_ref[...], axis=1)


def kernel(mem, strengths, idx, val, query):
    expect = jnp.arange(NWR, dtype=jnp.float32) + 2.0
    marker, valid = _sc_mark(idx, expect)
    wm = marker[:CAP].reshape(NB, 1, BLK)
    st = strengths.reshape(NB, 1, BLK)

    retrieved, tops, str3 = pl.pallas_call(
        _tc_body,
        grid=(NB,),
        in_specs=[
            pl.BlockSpec((NQ, D), lambda i: (0, 0)),
            pl.BlockSpec((NWR, D), lambda i: (0, 0)),
            pl.BlockSpec((1, NWR), lambda i: (0, 0)),
            pl.BlockSpec((BLK, D), lambda i: (i, 0)),
            pl.BlockSpec((1, 1, BLK), lambda i: (i, 0, 0)),
            pl.BlockSpec((1, 1, BLK), lambda i: (i, 0, 0)),
        ],
        out_specs=[
            pl.BlockSpec((NQ, D), lambda i: (0, 0)),
            pl.BlockSpec((NQ, K), lambda i: (0, 0)),
            pl.BlockSpec((1, 1, BLK), lambda i: (i, 0, 0)),
        ],
        out_shape=[
            jax.ShapeDtypeStruct((NQ, D), jnp.float32),
            jax.ShapeDtypeStruct((NQ, K), jnp.float32),
            jax.ShapeDtypeStruct((NB, 1, BLK), jnp.float32),
        ],
        scratch_shapes=[
            pltpu.VMEM((NQ, D), jnp.float32),
            pltpu.VMEM((NQ, K), jnp.float32),
            pltpu.VMEM((NQ, K, D), jnp.float32),
        ],
        compiler_params=pltpu.CompilerParams(
            dimension_semantics=("arbitrary",)),
    )(query, val, valid.reshape(1, NWR), mem, wm, st)

    return retrieved, tops, str3.reshape(CAP)


# speedup vs baseline: 2.5462x; 1.0241x over previous
import functools

import jax
import jax.numpy as jnp
from jax import lax
from jax.experimental import pallas as pl
from jax.experimental.pallas import tpu as pltpu
from jax.experimental.pallas import tpu_sc as plsc

CAP = 1_000_000
D = 64
K = 8
NWR = 16384
NQ = 16
DECAY = 0.02

SP = 1_000_448
CH = SP // 16
QCH = CH // 4
WCH = NWR // 16

BLK = 8000
NB = CAP // BLK
NEG = -1e30


def _sc_mark(idx, expect):
    mesh = plsc.VectorSubcoreMesh(core_axis_name="c", subcore_axis_name="s")

    @functools.partial(
        pl.kernel,
        mesh=mesh,
        out_type=[
            jax.ShapeDtypeStruct((SP,), jnp.float32),
            jax.ShapeDtypeStruct((NWR,), jnp.float32),
        ],
        scratch_types=[
            pltpu.VMEM((QCH,), jnp.float32),
            pltpu.VMEM((NWR,), jnp.int32),
            pltpu.VMEM((NWR,), jnp.float32),
            pltpu.VMEM((WCH,), jnp.int32),
            pltpu.VMEM((WCH,), jnp.float32),
            pltpu.VMEM((WCH,), jnp.float32),
            pltpu.SemaphoreType.DMA,
        ],
    )
    def k(idx_hbm, exp_hbm, ord_hbm, valid_hbm,
          zbuf, idx_all, exp_all, idxc, gbuf, ebuf, sem):
        cid = lax.axis_index("c")
        sid = lax.axis_index("s")

        @pl.when(cid == 0)
        def _work():
            def zb(t, c):
                zbuf[pl.ds(t * 16, 16)] = jnp.zeros((16,), jnp.float32)
                return c
            lax.fori_loop(0, QCH // 16, zb, 0)
            base = sid * CH

            def cq(q, c):
                pltpu.sync_copy(zbuf, ord_hbm.at[pl.ds(base + q * QCH, QCH)])
                return c
            lax.fori_loop(0, 4, cq, 0)
            plsc.subcore_barrier()

            @pl.when(sid == 0)
            def _scatter():
                pltpu.sync_copy(idx_hbm, idx_all)
                pltpu.sync_copy(exp_hbm, exp_all)
                pltpu.async_copy(exp_all, ord_hbm.at[idx_all], sem).wait()
            plsc.subcore_barrier()

            wbase = sid * WCH
            pltpu.sync_copy(idx_hbm.at[pl.ds(wbase, WCH)], idxc)
            pltpu.async_copy(ord_hbm.at[idxc], gbuf, sem).wait()
            pltpu.sync_copy(exp_hbm.at[pl.ds(wbase, WCH)], ebuf)

            def vv(t, c):
                g = gbuf[pl.ds(t * 16, 16)]
                e = ebuf[pl.ds(t * 16, 16)]
                ebuf[pl.ds(t * 16, 16)] = jnp.where(
                    g == e, jnp.float32(1.0), jnp.float32(0.0))
                return c
            lax.fori_loop(0, WCH // 16, vv, 0)
            pltpu.sync_copy(ebuf, valid_hbm.at[pl.ds(wbase, WCH)])

    return k(idx, expect)


def _tc_body(query_ref, val_ref, valid_ref, mem_ref, wm_ref, st_ref,
             retr_ref, tops_ref, str3_ref, qn_ref, rv_ref, rr_ref):
    i = pl.program_id(0)
    wm = wm_ref[0, 0, :]
    str3_ref[0, 0, :] = jnp.where(
        wm > 0.5, jnp.float32(1.0 - DECAY), st_ref[0, 0, :] * (1.0 - DECAY))

    def extract8(ext, rows_of_bulk):
        iota = lax.broadcasted_iota(jnp.int32, ext.shape, 1)
        vals, ohs = [], []
        for _ in range(K):
            m = jnp.max(ext, axis=1)
            am = jnp.min(jnp.where(ext == m[:, None], iota, jnp.int32(1 << 30)),
                         axis=1)
            ohs.append((iota == am[:, None]).astype(jnp.float32))
            vals.append(m)
            ext = jnp.where(iota == am[:, None], NEG, ext)
        rows3 = rows_of_bulk(jnp.concatenate(ohs, axis=0))
        rv_ref[...] = jnp.stack(vals, axis=1)
        rr_ref[...] = jnp.transpose(rows3, (1, 0, 2))

    @pl.when(i == 0)
    def _init():
        q = query_ref[...]
        qn = q / (jnp.sqrt(jnp.sum(q * q, axis=1, keepdims=True)) + 1e-12)
        qn_ref[...] = qn
        val = val_ref[...]
        scale = 1.0 / (jnp.sqrt(jnp.sum(val * val, axis=1)) + 1e-12)
        ws = jnp.dot(qn, val.T, preferred_element_type=jnp.float32)
        ws = ws * scale[None, :]
        ws = jnp.where(valid_ref[0, :][None, :] > 0.5, ws, NEG)
        mem0 = mem_ref[...]
        sims0 = jnp.dot(qn, mem0.T, preferred_element_type=jnp.float32)
        sims0 = jnp.where(wm[None, :] > 0.5, NEG, sims0)
        ext = jnp.concatenate([sims0, ws], axis=1)

        def rows_of_bulk(oh):
            rows = (jnp.dot(oh[:, :BLK], mem0,
                            preferred_element_type=jnp.float32)
                    + jnp.dot(oh[:, BLK:] * scale[None, :], val,
                              preferred_element_type=jnp.float32))
            return rows.reshape(K, NQ, D)
        extract8(ext, rows_of_bulk)

    @pl.when(i > 0)
    def _scan():
        qn = qn_ref[...]
        memb = mem_ref[...]
        sims = jnp.dot(qn, memb.T, preferred_element_type=jnp.float32)
        sims = jnp.where(wm[None, :] > 0.5, NEG, sims)
        prev_v = rv_ref[...]
        prev_r = rr_ref[...]
        ext = jnp.concatenate([prev_v, sims], axis=1)

        def rows_of_bulk(oh):
            rows = jnp.dot(oh[:, K:], memb,
                           preferred_element_type=jnp.float32).reshape(K, NQ, D)
            oh_r = oh[:, :K].reshape(K, NQ, K)
            for t in range(K):
                rows = rows + oh_r[:, :, t][:, :, None] * prev_r[None, :, t, :]
            return rows
        extract8(ext, rows_of_bulk)

    @pl.when(i == NB - 1)
    def _fin():
        tv = rv_ref[...]
        e = jnp.exp(tv - jnp.max(tv, axis=1, keepdims=True))
        w = e / jnp.sum(e, axis=1, keepdims=True)
        tops_ref[...] = tv
        retr_ref[...] = jnp.sum(w[:, :, None] * rr_ref[...], axis=1)


def kernel(mem, strengths, idx, val, query):
    expect = jnp.arange(NWR, dtype=jnp.float32) + 2.0
    marker, valid = _sc_mark(idx, expect)
    wm = marker[:CAP].reshape(NB, 1, BLK)
    st = strengths.reshape(NB, 1, BLK)

    retrieved, tops, str3 = pl.pallas_call(
        _tc_body,
        grid=(NB,),
        in_specs=[
            pl.BlockSpec((NQ, D), lambda i: (0, 0)),
            pl.BlockSpec((NWR, D), lambda i: (0, 0)),
            pl.BlockSpec((1, NWR), lambda i: (0, 0)),
            pl.BlockSpec((BLK, D), lambda i: (i, 0)),
            pl.BlockSpec((1, 1, BLK), lambda i: (i, 0, 0)),
            pl.BlockSpec((1, 1, BLK), lambda i: (i, 0, 0)),
        ],
        out_specs=[
            pl.BlockSpec((NQ, D), lambda i: (0, 0)),
            pl.BlockSpec((NQ, K), lambda i: (0, 0)),
            pl.BlockSpec((1, 1, BLK), lambda i: (i, 0, 0)),
        ],
        out_shape=[
            jax.ShapeDtypeStruct((NQ, D), jnp.float32),
            jax.ShapeDtypeStruct((NQ, K), jnp.float32),
            jax.ShapeDtypeStruct((NB, 1, BLK), jnp.float32),
        ],
        scratch_shapes=[
            pltpu.VMEM((NQ, D), jnp.float32),
            pltpu.VMEM((NQ, K), jnp.float32),
            pltpu.VMEM((NQ, K, D), jnp.float32),
        ],
        compiler_params=pltpu.CompilerParams(
            dimension_semantics=("arbitrary",)),
    )(query, val, valid.reshape(1, NWR), mem, wm, st)

    return retrieved, tops, str3.reshape(CAP)
